# baseline (speedup 1.0000x reference)
import functools

import jax
from jax import lax
import jax.numpy as jnp
from jax.experimental import pallas as pl
from jax.experimental.pallas import tpu as pltpu
from jax.experimental.pallas import tpu_sc as plsc

N = 10000
E = 320000
D = 128
NG = 64
DT = 10
EPS = 1e-5

NC = 2
NS = 16
NW = NC * NS
CH = 128
CPW = 80
EPW = CH * CPW
EPAD = EPW * NW
NCHUNK = EPAD // CH
NPAD = 10112
RPS = NPAD // NS

_mesh = plsc.VectorSubcoreMesh(core_axis_name="c", subcore_axis_name="s")


HCPW = CPW // 2


@functools.partial(
    pl.kernel,
    out_type=jax.ShapeDtypeStruct((NC, NPAD, D), jnp.float32),
    mesh=_mesh,
    scratch_types=[
        pltpu.VMEM((HCPW, CH), jnp.int32),
        pltpu.VMEM((HCPW, CH), jnp.int32),
        pltpu.VMEM((CH, D), jnp.float32),
        pltpu.VMEM((CH, D), jnp.float32),
        pltpu.VMEM_SHARED((NPAD, D), jnp.float32),
        pltpu.SemaphoreType.DMA,
        pltpu.SemaphoreType.DMA,
        pltpu.SemaphoreType.DMA,
        pltpu.SemaphoreType.DMA,
    ],
)
def _sc_scatter(hp_hbm, src_hbm, dst_hbm, zero_hbm, out_hbm,
                sidx, didx, rows0, rows1, acc, semg0, semg1, sems0, sems1):
    c = lax.axis_index("c")
    s = lax.axis_index("s")
    wid = s * NC + c
    pltpu.sync_copy(zero_hbm.at[pl.ds(s * RPS, RPS)], acc.at[pl.ds(s * RPS, RPS)])
    plsc.subcore_barrier()

    for half in range(2):
        cbase = wid * CPW + half * HCPW
        pltpu.sync_copy(src_hbm.at[pl.ds(cbase, HCPW)], sidx)
        pltpu.sync_copy(dst_hbm.at[pl.ds(cbase, HCPW)], didx)
        pltpu.async_copy(hp_hbm.at[sidx.at[0]], rows0, semg0)

        @pl.loop(0, HCPW, step=2)
        def _(j):
            @pl.when(j > 0)
            def _():
                pltpu.make_async_copy(rows1, acc.at[didx.at[j]], sems1).wait()

            pltpu.async_copy(hp_hbm.at[sidx.at[j + 1]], rows1, semg1)
            pltpu.make_async_copy(hp_hbm.at[sidx.at[j]], rows0, semg0).wait()
            pltpu.async_copy(rows0, acc.at[didx.at[j]], sems0, add=True)
            pltpu.make_async_copy(hp_hbm.at[sidx.at[j + 1]], rows1, semg1).wait()
            pltpu.async_copy(rows1, acc.at[didx.at[j + 1]], sems1, add=True)

            @pl.when(j + 2 < HCPW)
            def _():
                pltpu.make_async_copy(rows0, acc.at[didx.at[j]], sems0).wait()
                pltpu.async_copy(hp_hbm.at[sidx.at[j + 2]], rows0, semg0)

        pltpu.make_async_copy(rows0, acc.at[didx.at[0]], sems0).wait()
        pltpu.make_async_copy(rows1, acc.at[didx.at[0]], sems1).wait()

    plsc.subcore_barrier()
    pltpu.sync_copy(acc.at[pl.ds(s * RPS, RPS)], out_hbm.at[c, pl.ds(s * RPS, RPS)])


_BLK = 1000


def _embed_body(x_ref, w_ref, b_ref, o_ref):
    o_ref[...] = (
        jnp.dot(x_ref[...], w_ref[...], preferred_element_type=jnp.float32)
        + b_ref[...]
    )


def _tc_embed(x, W, b):
    return pl.pallas_call(
        _embed_body,
        grid=(N // _BLK,),
        in_specs=[
            pl.BlockSpec((_BLK, D), lambda i: (i, 0)),
            pl.BlockSpec((D, D), lambda i: (0, 0)),
            pl.BlockSpec((1, D), lambda i: (0, 0)),
        ],
        out_specs=pl.BlockSpec((_BLK, D), lambda i: (i, 0)),
        out_shape=jax.ShapeDtypeStruct((N, D), jnp.float32),
    )(x, W, b)


def _mms_body(h_ref, w_ref, c_ref, o_ref):
    dinv = lax.rsqrt(c_ref[0, :, 0:1] + c_ref[1, :, 0:1] + 1.0)
    o_ref[...] = (
        jnp.dot(h_ref[...], w_ref[...], preferred_element_type=jnp.float32) * dinv
    )


def _tc_matmul_scale(h, W, cnt):
    return pl.pallas_call(
        _mms_body,
        grid=(N // _BLK,),
        in_specs=[
            pl.BlockSpec((_BLK, D), lambda i: (i, 0)),
            pl.BlockSpec((D, D), lambda i: (0, 0)),
            pl.BlockSpec((NC, _BLK, D), lambda i: (0, i, 0)),
        ],
        out_specs=pl.BlockSpec((_BLK, D), lambda i: (i, 0)),
        out_shape=jax.ShapeDtypeStruct((N, D), jnp.float32),
    )(h, W, cnt)


def _post_body(h_ref, hp_ref, acc_ref, c_ref, b_ref, g_ref, be_ref, o_ref):
    dinv = lax.rsqrt(c_ref[0, :N, 0:1] + c_ref[1, :N, 0:1] + 1.0)
    t = (acc_ref[0, :N, :] + acc_ref[1, :N, :] + hp_ref[...]) * dinv + b_ref[...]
    mu = jnp.mean(t, axis=0, keepdims=True)
    var = jnp.mean((t - mu) ** 2, axis=0, keepdims=True)
    t = (t - mu) * lax.rsqrt(var + EPS) * g_ref[...] + be_ref[...]
    o_ref[...] = h_ref[...] + jnp.maximum(t, 0.0)


def _tc_post(h, hp, accp, cnt, b, g, be):
    return pl.pallas_call(
        _post_body,
        out_shape=jax.ShapeDtypeStruct((N, D), jnp.float32),
    )(h, hp, accp, cnt, b, g, be)


def _pool_body(h_ref, batch_ref, w1_ref, b1_ref, w2_ref, b2_ref, w3_ref,
               b3_ref, o_ref):
    gids = lax.broadcasted_iota(jnp.int32, (NG, N), 0)
    mask = (batch_ref[...] == gids).astype(jnp.float32)
    sums = jnp.dot(mask, h_ref[...], preferred_element_type=jnp.float32)
    cnt = jnp.sum(mask, axis=1, keepdims=True)
    pooled = sums / jnp.maximum(cnt, 1.0)
    z = jnp.maximum(
        jnp.dot(pooled, w1_ref[...], preferred_element_type=jnp.float32)
        + b1_ref[...], 0.0)
    z = jnp.maximum(
        jnp.dot(z, w2_ref[...], preferred_element_type=jnp.float32)
        + b2_ref[...], 0.0)
    o_ref[...] = (
        jnp.dot(z, w3_ref[...], preferred_element_type=jnp.float32) + b3_ref[...]
    )


def _tc_pool_mlp(h, batch2d, W1, b1, W2, b2, W3p, b3p):
    return pl.pallas_call(
        _pool_body,
        out_shape=jax.ShapeDtypeStruct((NG, D), jnp.float32),
    )(h, batch2d, W1, b1, W2, b2, W3p, b3p)


def kernel(x, edge_index, batch, W_emb, b_emb, W_conv0, b_conv0, gamma0,
           beta0, W_conv1, b_conv1, gamma1, beta1, W_conv2, b_conv2, gamma2,
           beta2, W_fc1, b_fc1, W_fc2, b_fc2, W_fc3, b_fc3):
    f32 = jnp.float32
    src = edge_index[0]
    dst = edge_index[1]
    pad = EPAD - E
    dump = N + (jnp.arange(pad, dtype=jnp.int32) % (NPAD - N))
    spread = jnp.arange(pad, dtype=jnp.int32) % N
    src2 = jnp.concatenate([src, spread]).reshape(NCHUNK, CH)
    dst2 = jnp.concatenate([dst, dump]).reshape(NCHUNK, CH)
    zeroD = jnp.zeros((NPAD, D), f32)
    onesN = jnp.ones((N, D), f32)

    cnt = _sc_scatter(onesN, src2, dst2, zeroD)
    h = _tc_embed(x, W_emb, b_emb.reshape(1, D))

    for (W, b, g, be) in (
        (W_conv0, b_conv0, gamma0, beta0),
        (W_conv1, b_conv1, gamma1, beta1),
        (W_conv2, b_conv2, gamma2, beta2),
    ):
        hp = _tc_matmul_scale(h, W, cnt)
        accp = _sc_scatter(hp, src2, dst2, zeroD)
        h = _tc_post(h, hp, accp, cnt, b.reshape(1, D), g.reshape(1, D),
                     be.reshape(1, D))

    W3p = jnp.pad(W_fc3, ((0, 0), (0, D - DT)))
    b3p = jnp.pad(b_fc3, (0, D - DT)).reshape(1, D)
    zp = _tc_pool_mlp(h, batch.reshape(1, N), W_fc1, b_fc1.reshape(1, D // 2),
                      W_fc2, b_fc2.reshape(1, D // 4), W3p, b3p)
    return zp[:, :DT]


# speedup vs baseline: 1.3047x; 1.3047x over previous
import functools

import jax
from jax import lax
import jax.numpy as jnp
from jax.experimental import pallas as pl
from jax.experimental.pallas import tpu as pltpu
from jax.experimental.pallas import tpu_sc as plsc

N = 10000
E = 320000
D = 128
NG = 64
DT = 10
EPS = 1e-5

NC = 2
NS = 16
NW = NC * NS
CH = 128
CPW = 80
EPW = CH * CPW
EPAD = EPW * NW
NCHUNK = EPAD // CH
NPAD = 10112
RPS = NPAD // NS

_mesh = plsc.VectorSubcoreMesh(core_axis_name="c", subcore_axis_name="s")


HCPW = CPW // 2


@functools.partial(
    pl.kernel,
    out_type=jax.ShapeDtypeStruct((NC, NPAD, D), jnp.float32),
    mesh=_mesh,
    scratch_types=[
        pltpu.VMEM((HCPW, CH), jnp.int32),
        pltpu.VMEM((HCPW, CH), jnp.int32),
        pltpu.VMEM((CH, D), jnp.float32),
        pltpu.VMEM((CH, D), jnp.float32),
        pltpu.VMEM_SHARED((NPAD, D), jnp.float32),
        pltpu.SemaphoreType.DMA,
        pltpu.SemaphoreType.DMA,
    ],
)
def _sc_scatter(hp_hbm, src_hbm, dst_hbm, zero_hbm, out_hbm,
                sidx, didx, rows0, rows1, acc, sem0, sem1):
    c = lax.axis_index("c")
    s = lax.axis_index("s")
    wid = s * NC + c
    pltpu.sync_copy(zero_hbm.at[pl.ds(s * RPS, RPS)], acc.at[pl.ds(s * RPS, RPS)])
    plsc.subcore_barrier()

    for half in range(2):
        cbase = wid * CPW + half * HCPW
        pltpu.sync_copy(src_hbm.at[pl.ds(cbase, HCPW)], sidx)
        pltpu.sync_copy(dst_hbm.at[pl.ds(cbase, HCPW)], didx)
        pltpu.async_copy(hp_hbm.at[sidx.at[0]], rows0, sem0)

        @pl.loop(0, HCPW, step=2)
        def _(j):
            pltpu.async_copy(hp_hbm.at[sidx.at[j + 1]], rows1, sem1)
            pltpu.make_async_copy(hp_hbm.at[sidx.at[j]], rows0, sem0).wait()
            pltpu.sync_copy(rows0, acc.at[didx.at[j]], add=True)

            @pl.when(j + 2 < HCPW)
            def _():
                pltpu.async_copy(hp_hbm.at[sidx.at[j + 2]], rows0, sem0)

            pltpu.make_async_copy(hp_hbm.at[sidx.at[j + 1]], rows1, sem1).wait()
            pltpu.sync_copy(rows1, acc.at[didx.at[j + 1]], add=True)

    plsc.subcore_barrier()
    pltpu.sync_copy(acc.at[pl.ds(s * RPS, RPS)], out_hbm.at[c, pl.ds(s * RPS, RPS)])


_BLK = 1000


def _embed_mms_body(x_ref, we_ref, be_ref, w0_ref, c_ref, h_ref, hp_ref):
    h = (
        jnp.dot(x_ref[...], we_ref[...], preferred_element_type=jnp.float32)
        + be_ref[...]
    )
    h_ref[...] = h
    dinv = lax.rsqrt(c_ref[0, :, 0:1] + c_ref[1, :, 0:1] + 1.0)
    hp_ref[...] = (
        jnp.dot(h, w0_ref[...], preferred_element_type=jnp.float32) * dinv
    )


def _tc_embed_mms(x, We, be, W0, cnt):
    return pl.pallas_call(
        _embed_mms_body,
        grid=(N // _BLK,),
        in_specs=[
            pl.BlockSpec((_BLK, D), lambda i: (i, 0)),
            pl.BlockSpec((D, D), lambda i: (0, 0)),
            pl.BlockSpec((1, D), lambda i: (0, 0)),
            pl.BlockSpec((D, D), lambda i: (0, 0)),
            pl.BlockSpec((NC, _BLK, D), lambda i: (0, i, 0)),
        ],
        out_specs=[
            pl.BlockSpec((_BLK, D), lambda i: (i, 0)),
            pl.BlockSpec((_BLK, D), lambda i: (i, 0)),
        ],
        out_shape=[
            jax.ShapeDtypeStruct((N, D), jnp.float32),
            jax.ShapeDtypeStruct((N, D), jnp.float32),
        ],
    )(x, We, be, W0, cnt)


def _post_core(h_ref, hp_ref, acc_ref, c_ref, b_ref, g_ref, be_ref):
    dinv = lax.rsqrt(c_ref[0, :N, 0:1] + c_ref[1, :N, 0:1] + 1.0)
    t = (acc_ref[0, :N, :] + acc_ref[1, :N, :] + hp_ref[...]) * dinv + b_ref[...]
    mu = jnp.mean(t, axis=0, keepdims=True)
    var = jnp.mean((t - mu) ** 2, axis=0, keepdims=True)
    t = (t - mu) * lax.rsqrt(var + EPS) * g_ref[...] + be_ref[...]
    return h_ref[...] + jnp.maximum(t, 0.0), dinv


def _post_mms_body(h_ref, hp_ref, acc_ref, c_ref, b_ref, g_ref, be_ref,
                   wn_ref, o_ref, hpn_ref):
    h_new, dinv = _post_core(h_ref, hp_ref, acc_ref, c_ref, b_ref, g_ref, be_ref)
    o_ref[...] = h_new
    hpn_ref[...] = (
        jnp.dot(h_new, wn_ref[...], preferred_element_type=jnp.float32) * dinv
    )


def _tc_post_mms(h, hp, accp, cnt, b, g, be, Wn):
    return pl.pallas_call(
        _post_mms_body,
        out_shape=[
            jax.ShapeDtypeStruct((N, D), jnp.float32),
            jax.ShapeDtypeStruct((N, D), jnp.float32),
        ],
    )(h, hp, accp, cnt, b, g, be, Wn)


def _post_body(h_ref, hp_ref, acc_ref, c_ref, b_ref, g_ref, be_ref, o_ref):
    o_ref[...] = _post_core(h_ref, hp_ref, acc_ref, c_ref, b_ref, g_ref,
                            be_ref)[0]


def _tc_post(h, hp, accp, cnt, b, g, be):
    return pl.pallas_call(
        _post_body,
        out_shape=jax.ShapeDtypeStruct((N, D), jnp.float32),
    )(h, hp, accp, cnt, b, g, be)


def _pool_body(h_ref, batch_ref, w1_ref, b1_ref, w2_ref, b2_ref, w3_ref,
               b3_ref, o_ref):
    gids = lax.broadcasted_iota(jnp.int32, (NG, N), 0)
    mask = (batch_ref[...] == gids).astype(jnp.float32)
    sums = jnp.dot(mask, h_ref[...], preferred_element_type=jnp.float32)
    cnt = jnp.sum(mask, axis=1, keepdims=True)
    pooled = sums / jnp.maximum(cnt, 1.0)
    z = jnp.maximum(
        jnp.dot(pooled, w1_ref[...], preferred_element_type=jnp.float32)
        + b1_ref[...], 0.0)
    z = jnp.maximum(
        jnp.dot(z, w2_ref[...], preferred_element_type=jnp.float32)
        + b2_ref[...], 0.0)
    o_ref[...] = (
        jnp.dot(z, w3_ref[...], preferred_element_type=jnp.float32) + b3_ref[...]
    )


def _tc_pool_mlp(h, batch2d, W1, b1, W2, b2, W3p, b3p):
    return pl.pallas_call(
        _pool_body,
        out_shape=jax.ShapeDtypeStruct((NG, D), jnp.float32),
    )(h, batch2d, W1, b1, W2, b2, W3p, b3p)


def kernel(x, edge_index, batch, W_emb, b_emb, W_conv0, b_conv0, gamma0,
           beta0, W_conv1, b_conv1, gamma1, beta1, W_conv2, b_conv2, gamma2,
           beta2, W_fc1, b_fc1, W_fc2, b_fc2, W_fc3, b_fc3):
    f32 = jnp.float32
    src = edge_index[0]
    dst = edge_index[1]
    pad = EPAD - E
    dump = N + (jnp.arange(pad, dtype=jnp.int32) % (NPAD - N))
    spread = jnp.arange(pad, dtype=jnp.int32) % N
    src2 = jnp.concatenate([src, spread]).reshape(NCHUNK, CH)
    dst2 = jnp.concatenate([dst, dump]).reshape(NCHUNK, CH)
    zeroD = jnp.zeros((NPAD, D), f32)
    onesN = jnp.ones((N, D), f32)

    cnt = _sc_scatter(onesN, src2, dst2, zeroD)
    h, hp = _tc_embed_mms(x, W_emb, b_emb.reshape(1, D), W_conv0, cnt)

    layers = (
        (b_conv0, gamma0, beta0, W_conv1),
        (b_conv1, gamma1, beta1, W_conv2),
        (b_conv2, gamma2, beta2, None),
    )
    for (b, g, be, Wn) in layers:
        accp = _sc_scatter(hp, src2, dst2, zeroD)
        if Wn is not None:
            h, hp = _tc_post_mms(h, hp, accp, cnt, b.reshape(1, D),
                                 g.reshape(1, D), be.reshape(1, D), Wn)
        else:
            h = _tc_post(h, hp, accp, cnt, b.reshape(1, D), g.reshape(1, D),
                         be.reshape(1, D))

    W3p = jnp.pad(W_fc3, ((0, 0), (0, D - DT)))
    b3p = jnp.pad(b_fc3, (0, D - DT)).reshape(1, D)
    zp = _tc_pool_mlp(h, batch.reshape(1, N), W_fc1, b_fc1.reshape(1, D // 2),
                      W_fc2, b_fc2.reshape(1, D // 4), W3p, b3p)
    return zp[:, :DT]


# speedup vs baseline: 1.4081x; 1.0793x over previous
import functools

import jax
from jax import lax
import jax.numpy as jnp
from jax.experimental import pallas as pl
from jax.experimental.pallas import tpu as pltpu
from jax.experimental.pallas import tpu_sc as plsc

N = 10000
E = 320000
D = 128
NG = 64
DT = 10
EPS = 1e-5

NC = 2
NS = 16
NW = NC * NS
CH = 128
CPW = 80
EPW = CH * CPW
EPAD = EPW * NW
NCHUNK = EPAD // CH
NPAD = 10112
RPS = NPAD // NS

_mesh = plsc.VectorSubcoreMesh(core_axis_name="c", subcore_axis_name="s")


HCPW = CPW // 2


@functools.partial(
    pl.kernel,
    out_type=jax.ShapeDtypeStruct((NC, NPAD, D), jnp.float32),
    mesh=_mesh,
    scratch_types=[
        pltpu.VMEM((HCPW, CH), jnp.int32),
        pltpu.VMEM((CH, D), jnp.float32),
        pltpu.VMEM_SHARED((NPAD, D), jnp.float32),
    ],
)
def _sc_count(dst_hbm, ones_hbm, zero_hbm, out_hbm, didx, ones_v, cnt):
    c = lax.axis_index("c")
    s = lax.axis_index("s")
    wid = s * NC + c
    pltpu.sync_copy(zero_hbm.at[pl.ds(s * RPS, RPS)], cnt.at[pl.ds(s * RPS, RPS)])
    pltpu.sync_copy(ones_hbm, ones_v)
    plsc.subcore_barrier()

    for half in range(2):
        cbase = wid * CPW + half * HCPW
        pltpu.sync_copy(dst_hbm.at[pl.ds(cbase, HCPW)], didx)

        @pl.loop(0, HCPW)
        def _(j):
            pltpu.sync_copy(ones_v, cnt.at[didx.at[j]], add=True)

    plsc.subcore_barrier()
    pltpu.sync_copy(cnt.at[pl.ds(s * RPS, RPS)], out_hbm.at[c, pl.ds(s * RPS, RPS)])


@functools.partial(
    pl.kernel,
    out_type=jax.ShapeDtypeStruct((NC, NPAD, D), jnp.float32),
    mesh=_mesh,
    scratch_types=[
        pltpu.VMEM((HCPW, CH), jnp.int32),
        pltpu.VMEM((HCPW, CH), jnp.int32),
        pltpu.VMEM((CH, D), jnp.float32),
        pltpu.VMEM((CH, D), jnp.float32),
        pltpu.VMEM_SHARED((NPAD, D), jnp.float32),
        pltpu.SemaphoreType.DMA,
        pltpu.SemaphoreType.DMA,
    ],
)
def _sc_scatter(hp_hbm, src_hbm, dst_hbm, zero_hbm, out_hbm,
                sidx, didx, rows0, rows1, acc, sem0, sem1):
    c = lax.axis_index("c")
    s = lax.axis_index("s")
    wid = s * NC + c
    pltpu.sync_copy(zero_hbm.at[pl.ds(s * RPS, RPS)], acc.at[pl.ds(s * RPS, RPS)])
    plsc.subcore_barrier()

    for half in range(2):
        cbase = wid * CPW + half * HCPW
        pltpu.sync_copy(src_hbm.at[pl.ds(cbase, HCPW)], sidx)
        pltpu.sync_copy(dst_hbm.at[pl.ds(cbase, HCPW)], didx)
        pltpu.async_copy(hp_hbm.at[sidx.at[0]], rows0, sem0)

        @pl.loop(0, HCPW, step=2)
        def _(j):
            pltpu.async_copy(hp_hbm.at[sidx.at[j + 1]], rows1, sem1)
            pltpu.make_async_copy(hp_hbm.at[sidx.at[j]], rows0, sem0).wait()
            pltpu.sync_copy(rows0, acc.at[didx.at[j]], add=True)

            @pl.when(j + 2 < HCPW)
            def _():
                pltpu.async_copy(hp_hbm.at[sidx.at[j + 2]], rows0, sem0)

            pltpu.make_async_copy(hp_hbm.at[sidx.at[j + 1]], rows1, sem1).wait()
            pltpu.sync_copy(rows1, acc.at[didx.at[j + 1]], add=True)

    plsc.subcore_barrier()
    pltpu.sync_copy(acc.at[pl.ds(s * RPS, RPS)], out_hbm.at[c, pl.ds(s * RPS, RPS)])


_BLK = 1000


def _embed_mms_body(x_ref, we_ref, be_ref, w0_ref, c_ref, h_ref, hp_ref):
    h = (
        jnp.dot(x_ref[...], we_ref[...], preferred_element_type=jnp.float32)
        + be_ref[...]
    )
    h_ref[...] = h
    dinv = lax.rsqrt(c_ref[0, :, 0:1] + c_ref[1, :, 0:1] + 1.0)
    hp_ref[...] = (
        jnp.dot(h, w0_ref[...], preferred_element_type=jnp.float32) * dinv
    )


def _tc_embed_mms(x, We, be, W0, cnt):
    return pl.pallas_call(
        _embed_mms_body,
        grid=(N // _BLK,),
        in_specs=[
            pl.BlockSpec((_BLK, D), lambda i: (i, 0)),
            pl.BlockSpec((D, D), lambda i: (0, 0)),
            pl.BlockSpec((1, D), lambda i: (0, 0)),
            pl.BlockSpec((D, D), lambda i: (0, 0)),
            pl.BlockSpec((NC, _BLK, D), lambda i: (0, i, 0)),
        ],
        out_specs=[
            pl.BlockSpec((_BLK, D), lambda i: (i, 0)),
            pl.BlockSpec((_BLK, D), lambda i: (i, 0)),
        ],
        out_shape=[
            jax.ShapeDtypeStruct((N, D), jnp.float32),
            jax.ShapeDtypeStruct((N, D), jnp.float32),
        ],
    )(x, We, be, W0, cnt)


def _post_core(h_ref, hp_ref, acc_ref, c_ref, b_ref, g_ref, be_ref):
    dinv = lax.rsqrt(c_ref[0, :N, 0:1] + c_ref[1, :N, 0:1] + 1.0)
    t = (acc_ref[0, :N, :] + acc_ref[1, :N, :] + hp_ref[...]) * dinv + b_ref[...]
    mu = jnp.mean(t, axis=0, keepdims=True)
    var = jnp.mean((t - mu) ** 2, axis=0, keepdims=True)
    t = (t - mu) * lax.rsqrt(var + EPS) * g_ref[...] + be_ref[...]
    return h_ref[...] + jnp.maximum(t, 0.0), dinv


def _post_mms_body(h_ref, hp_ref, acc_ref, c_ref, b_ref, g_ref, be_ref,
                   wn_ref, o_ref, hpn_ref):
    h_new, dinv = _post_core(h_ref, hp_ref, acc_ref, c_ref, b_ref, g_ref, be_ref)
    o_ref[...] = h_new
    hpn_ref[...] = (
        jnp.dot(h_new, wn_ref[...], preferred_element_type=jnp.float32) * dinv
    )


def _tc_post_mms(h, hp, accp, cnt, b, g, be, Wn):
    return pl.pallas_call(
        _post_mms_body,
        out_shape=[
            jax.ShapeDtypeStruct((N, D), jnp.float32),
            jax.ShapeDtypeStruct((N, D), jnp.float32),
        ],
    )(h, hp, accp, cnt, b, g, be, Wn)


def _post_body(h_ref, hp_ref, acc_ref, c_ref, b_ref, g_ref, be_ref, o_ref):
    o_ref[...] = _post_core(h_ref, hp_ref, acc_ref, c_ref, b_ref, g_ref,
                            be_ref)[0]


def _tc_post(h, hp, accp, cnt, b, g, be):
    return pl.pallas_call(
        _post_body,
        out_shape=jax.ShapeDtypeStruct((N, D), jnp.float32),
    )(h, hp, accp, cnt, b, g, be)


def _pool_body(h_ref, batch_ref, w1_ref, b1_ref, w2_ref, b2_ref, w3_ref,
               b3_ref, o_ref):
    gids = lax.broadcasted_iota(jnp.int32, (NG, N), 0)
    mask = (batch_ref[...] == gids).astype(jnp.float32)
    sums = jnp.dot(mask, h_ref[...], preferred_element_type=jnp.float32)
    cnt = jnp.sum(mask, axis=1, keepdims=True)
    pooled = sums / jnp.maximum(cnt, 1.0)
    z = jnp.maximum(
        jnp.dot(pooled, w1_ref[...], preferred_element_type=jnp.float32)
        + b1_ref[...], 0.0)
    z = jnp.maximum(
        jnp.dot(z, w2_ref[...], preferred_element_type=jnp.float32)
        + b2_ref[...], 0.0)
    o_ref[...] = (
        jnp.dot(z, w3_ref[...], preferred_element_type=jnp.float32) + b3_ref[...]
    )


def _tc_pool_mlp(h, batch2d, W1, b1, W2, b2, W3p, b3p):
    return pl.pallas_call(
        _pool_body,
        out_shape=jax.ShapeDtypeStruct((NG, D), jnp.float32),
    )(h, batch2d, W1, b1, W2, b2, W3p, b3p)


def kernel(x, edge_index, batch, W_emb, b_emb, W_conv0, b_conv0, gamma0,
           beta0, W_conv1, b_conv1, gamma1, beta1, W_conv2, b_conv2, gamma2,
           beta2, W_fc1, b_fc1, W_fc2, b_fc2, W_fc3, b_fc3):
    f32 = jnp.float32
    src = edge_index[0]
    dst = edge_index[1]
    pad = EPAD - E
    dump = N + (jnp.arange(pad, dtype=jnp.int32) % (NPAD - N))
    spread = jnp.arange(pad, dtype=jnp.int32) % N
    src2 = jnp.concatenate([src, spread]).reshape(NCHUNK, CH)
    dst2 = jnp.concatenate([dst, dump]).reshape(NCHUNK, CH)
    zeroD = jnp.zeros((NPAD, D), f32)
    onesD = jnp.ones((CH, D), f32)

    cnt = _sc_count(dst2, onesD, zeroD)
    h, hp = _tc_embed_mms(x, W_emb, b_emb.reshape(1, D), W_conv0, cnt)

    layers = (
        (b_conv0, gamma0, beta0, W_conv1),
        (b_conv1, gamma1, beta1, W_conv2),
        (b_conv2, gamma2, beta2, None),
    )
    for (b, g, be, Wn) in layers:
        accp = _sc_scatter(hp, src2, dst2, zeroD)
        if Wn is not None:
            h, hp = _tc_post_mms(h, hp, accp, cnt, b.reshape(1, D),
                                 g.reshape(1, D), be.reshape(1, D), Wn)
        else:
            h = _tc_post(h, hp, accp, cnt, b.reshape(1, D), g.reshape(1, D),
                         be.reshape(1, D))

    W3p = jnp.pad(W_fc3, ((0, 0), (0, D - DT)))
    b3p = jnp.pad(b_fc3, (0, D - DT)).reshape(1, D)
    zp = _tc_pool_mlp(h, batch.reshape(1, N), W_fc1, b_fc1.reshape(1, D // 2),
                      W_fc2, b_fc2.reshape(1, D // 4), W3p, b3p)
    return zp[:, :DT]


# speedup vs baseline: 1.4186x; 1.0075x over previous
import functools

import jax
from jax import lax
import jax.numpy as jnp
from jax.experimental import pallas as pl
from jax.experimental.pallas import tpu as pltpu
from jax.experimental.pallas import tpu_sc as plsc

N = 10000
E = 320000
D = 128
NG = 64
DT = 10
EPS = 1e-5

NC = 2
NS = 16
NW = NC * NS
CH = 128
CPW = 80
EPW = CH * CPW
EPAD = EPW * NW
NCHUNK = EPAD // CH
NPAD = 10112
RPS = NPAD // NS

_mesh = plsc.VectorSubcoreMesh(core_axis_name="c", subcore_axis_name="s")


HCPW = CPW // 2


@functools.partial(
    pl.kernel,
    out_type=jax.ShapeDtypeStruct((NC, NPAD, D), jnp.float32),
    mesh=_mesh,
    scratch_types=[
        pltpu.VMEM((HCPW, CH), jnp.int32),
        pltpu.VMEM((CH, D), jnp.float32),
        pltpu.VMEM_SHARED((NPAD, D), jnp.float32),
    ],
)
def _sc_count(dst_hbm, ones_hbm, zero_hbm, out_hbm, didx, ones_v, cnt):
    c = lax.axis_index("c")
    s = lax.axis_index("s")
    wid = s * NC + c
    pltpu.sync_copy(zero_hbm.at[pl.ds(s * RPS, RPS)], cnt.at[pl.ds(s * RPS, RPS)])
    pltpu.sync_copy(ones_hbm, ones_v)
    plsc.subcore_barrier()

    for half in range(2):
        cbase = wid * CPW + half * HCPW
        pltpu.sync_copy(dst_hbm.at[pl.ds(cbase, HCPW)], didx)

        @pl.loop(0, HCPW)
        def _(j):
            pltpu.sync_copy(ones_v, cnt.at[didx.at[j]], add=True)

    plsc.subcore_barrier()
    pltpu.sync_copy(cnt.at[pl.ds(s * RPS, RPS)], out_hbm.at[c, pl.ds(s * RPS, RPS)])


@functools.partial(
    pl.kernel,
    out_type=jax.ShapeDtypeStruct((NC, NPAD, D), jnp.float32),
    mesh=_mesh,
    scratch_types=[
        pltpu.VMEM((HCPW, CH), jnp.int32),
        pltpu.VMEM((HCPW, CH), jnp.int32),
        pltpu.VMEM((CH, D), jnp.float32),
        pltpu.VMEM((CH, D), jnp.float32),
        pltpu.VMEM_SHARED((NPAD, D), jnp.float32),
        pltpu.SemaphoreType.DMA,
        pltpu.SemaphoreType.DMA,
    ],
)
def _sc_scatter(hp_hbm, src_hbm, dst_hbm, zero_hbm, out_hbm,
                sidx, didx, rows0, rows1, acc, sem0, sem1):
    c = lax.axis_index("c")
    s = lax.axis_index("s")
    wid = s * NC + c
    pltpu.sync_copy(zero_hbm.at[pl.ds(s * RPS, RPS)], acc.at[pl.ds(s * RPS, RPS)])
    plsc.subcore_barrier()

    for half in range(2):
        cbase = wid * CPW + half * HCPW
        pltpu.sync_copy(src_hbm.at[pl.ds(cbase, HCPW)], sidx)
        pltpu.sync_copy(dst_hbm.at[pl.ds(cbase, HCPW)], didx)
        pltpu.async_copy(hp_hbm.at[sidx.at[0]], rows0, sem0)

        @pl.loop(0, HCPW, step=2)
        def _(j):
            pltpu.async_copy(hp_hbm.at[sidx.at[j + 1]], rows1, sem1)
            pltpu.make_async_copy(hp_hbm.at[sidx.at[j]], rows0, sem0).wait()
            pltpu.sync_copy(rows0, acc.at[didx.at[j]], add=True)

            @pl.when(j + 2 < HCPW)
            def _():
                pltpu.async_copy(hp_hbm.at[sidx.at[j + 2]], rows0, sem0)

            pltpu.make_async_copy(hp_hbm.at[sidx.at[j + 1]], rows1, sem1).wait()
            pltpu.sync_copy(rows1, acc.at[didx.at[j + 1]], add=True)

    plsc.subcore_barrier()
    pltpu.sync_copy(acc.at[pl.ds(s * RPS, RPS)], out_hbm.at[c, pl.ds(s * RPS, RPS)])


_BLK = 1000


def _embed_mms_body(x_ref, we_ref, be_ref, w0_ref, c_ref, h_ref, hp_ref):
    h = (
        jnp.dot(x_ref[...], we_ref[...], preferred_element_type=jnp.float32)
        + be_ref[...]
    )
    h_ref[...] = h
    dinv = lax.rsqrt(c_ref[0, :, 0:1] + c_ref[1, :, 0:1] + 1.0)
    hp_ref[...] = (
        jnp.dot(h, w0_ref[...], preferred_element_type=jnp.float32) * dinv
    )


def _tc_embed_mms(x, We, be, W0, cnt):
    return pl.pallas_call(
        _embed_mms_body,
        grid=(N // _BLK,),
        in_specs=[
            pl.BlockSpec((_BLK, D), lambda i: (i, 0)),
            pl.BlockSpec((D, D), lambda i: (0, 0)),
            pl.BlockSpec((1, D), lambda i: (0, 0)),
            pl.BlockSpec((D, D), lambda i: (0, 0)),
            pl.BlockSpec((NC, _BLK, D), lambda i: (0, i, 0)),
        ],
        out_specs=[
            pl.BlockSpec((_BLK, D), lambda i: (i, 0)),
            pl.BlockSpec((_BLK, D), lambda i: (i, 0)),
        ],
        out_shape=[
            jax.ShapeDtypeStruct((N, D), jnp.float32),
            jax.ShapeDtypeStruct((N, D), jnp.float32),
        ],
    )(x, We, be, W0, cnt)


def _post_core(h_ref, hp_ref, acc_ref, c_ref, b_ref, g_ref, be_ref):
    dinv = lax.rsqrt(c_ref[0, :N, 0:1] + c_ref[1, :N, 0:1] + 1.0)
    t = (acc_ref[0, :N, :] + acc_ref[1, :N, :] + hp_ref[...]) * dinv + b_ref[...]
    mu = jnp.mean(t, axis=0, keepdims=True)
    var = jnp.mean((t - mu) ** 2, axis=0, keepdims=True)
    t = (t - mu) * lax.rsqrt(var + EPS) * g_ref[...] + be_ref[...]
    return h_ref[...] + jnp.maximum(t, 0.0), dinv


def _post_mms_body(h_ref, hp_ref, acc_ref, c_ref, b_ref, g_ref, be_ref,
                   wn_ref, o_ref, hpn_ref):
    h_new, dinv = _post_core(h_ref, hp_ref, acc_ref, c_ref, b_ref, g_ref, be_ref)
    o_ref[...] = h_new
    hpn_ref[...] = (
        jnp.dot(h_new, wn_ref[...], preferred_element_type=jnp.float32) * dinv
    )


def _tc_post_mms(h, hp, accp, cnt, b, g, be, Wn):
    return pl.pallas_call(
        _post_mms_body,
        out_shape=[
            jax.ShapeDtypeStruct((N, D), jnp.float32),
            jax.ShapeDtypeStruct((N, D), jnp.float32),
        ],
    )(h, hp, accp, cnt, b, g, be, Wn)


def _post_pool_body(h_ref, hp_ref, acc_ref, c_ref, b_ref, g_ref, be_ref,
                    batch_ref, w1_ref, b1_ref, w2_ref, b2_ref, w3_ref,
                    b3_ref, o_ref):
    h_new = _post_core(h_ref, hp_ref, acc_ref, c_ref, b_ref, g_ref, be_ref)[0]
    gids = lax.broadcasted_iota(jnp.int32, (NG, N), 0)
    mask = (batch_ref[...] == gids).astype(jnp.float32)
    sums = jnp.dot(mask, h_new, preferred_element_type=jnp.float32)
    cnt = jnp.sum(mask, axis=1, keepdims=True)
    pooled = sums / jnp.maximum(cnt, 1.0)
    z = jnp.maximum(
        jnp.dot(pooled, w1_ref[...], preferred_element_type=jnp.float32)
        + b1_ref[...], 0.0)
    z = jnp.maximum(
        jnp.dot(z, w2_ref[...], preferred_element_type=jnp.float32)
        + b2_ref[...], 0.0)
    o_ref[...] = (
        jnp.dot(z, w3_ref[...], preferred_element_type=jnp.float32) + b3_ref[...]
    )


def _tc_post_pool(h, hp, accp, cnt, b, g, be, batch2d, W1, b1, W2, b2, W3p,
                  b3p):
    return pl.pallas_call(
        _post_pool_body,
        out_shape=jax.ShapeDtypeStruct((NG, D), jnp.float32),
    )(h, hp, accp, cnt, b, g, be, batch2d, W1, b1, W2, b2, W3p, b3p)


def kernel(x, edge_index, batch, W_emb, b_emb, W_conv0, b_conv0, gamma0,
           beta0, W_conv1, b_conv1, gamma1, beta1, W_conv2, b_conv2, gamma2,
           beta2, W_fc1, b_fc1, W_fc2, b_fc2, W_fc3, b_fc3):
    f32 = jnp.float32
    src = edge_index[0]
    dst = edge_index[1]
    pad = EPAD - E
    dump = N + (jnp.arange(pad, dtype=jnp.int32) % (NPAD - N))
    spread = jnp.arange(pad, dtype=jnp.int32) % N
    src2 = jnp.concatenate([src, spread]).reshape(NCHUNK, CH)
    dst2 = jnp.concatenate([dst, dump]).reshape(NCHUNK, CH)
    zeroD = jnp.zeros((NPAD, D), f32)
    onesD = jnp.ones((CH, D), f32)

    cnt = _sc_count(dst2, onesD, zeroD)
    h, hp = _tc_embed_mms(x, W_emb, b_emb.reshape(1, D), W_conv0, cnt)

    for (b, g, be, Wn) in (
        (b_conv0, gamma0, beta0, W_conv1),
        (b_conv1, gamma1, beta1, W_conv2),
    ):
        accp = _sc_scatter(hp, src2, dst2, zeroD)
        h, hp = _tc_post_mms(h, hp, accp, cnt, b.reshape(1, D),
                             g.reshape(1, D), be.reshape(1, D), Wn)

    accp = _sc_scatter(hp, src2, dst2, zeroD)
    W3p = jnp.pad(W_fc3, ((0, 0), (0, D - DT)))
    b3p = jnp.pad(b_fc3, (0, D - DT)).reshape(1, D)
    zp = _tc_post_pool(h, hp, accp, cnt, b_conv2.reshape(1, D),
                       gamma2.reshape(1, D), beta2.reshape(1, D),
                       batch.reshape(1, N), W_fc1, b_fc1.reshape(1, D // 2),
                       W_fc2, b_fc2.reshape(1, D // 4), W3p, b3p)
    return zp[:, :DT]
